# baseline (speedup 1.0000x reference)
import jax
import jax.numpy as jnp
from jax.experimental import pallas as pl

NUM_SELECT = 300
NUM_CLASSES = 80


def kernel(pred_logits, pred_boxes, target_sizes, positive_map):
    prob = jax.nn.sigmoid(pred_logits) @ positive_map.T
    B, Q, C = prob.shape
    topk_values, topk_indexes = jax.lax.top_k(prob.reshape(B, Q * C), NUM_SELECT)
    scores = topk_values
    topk_boxes = topk_indexes // C
    labels = topk_indexes % C
    cx, cy, w, h = jnp.split(pred_boxes, 4, axis=-1)
    boxes = jnp.concatenate(
        [cx - 0.5 * w, cy - 0.5 * h, cx + 0.5 * w, cy + 0.5 * h], axis=-1
    )
    gather_idx = jnp.repeat(topk_boxes[:, :, None], 4, axis=2)
    boxes = jnp.take_along_axis(boxes, gather_idx, axis=1)
    ts = target_sizes.astype(jnp.float32)
    scale_fct = jnp.stack([ts[:, 1], ts[:, 0], ts[:, 1], ts[:, 0]], axis=1)
    boxes = boxes * scale_fct[:, None, :]
    return scores, labels, boxes


# speedup vs baseline: 2.8070x; 2.8070x over previous
import functools

import jax
import jax.numpy as jnp
from jax import lax
from jax.experimental import pallas as pl
from jax.experimental.pallas import tpu as pltpu
from jax.experimental.pallas import tpu_sc as plsc

B = 16
Q = 900
C = 80
TOK = 256
K = 300
HALF = Q * C // 2
NV = HALF // 16
NB = 4096
NBV = NB // 16
CAND = 384
M = 2 * CAND
R = 384

_HI = lax.Precision.HIGHEST


def _prob_body(logits_ref, pmt_ref, keys_ref):
    sig = jax.nn.sigmoid(logits_ref[0])
    prob = jnp.dot(sig, pmt_ref[...],
                   preferred_element_type=jnp.float32)
    keys_ref[0] = lax.bitcast_convert_type(prob, jnp.int32)


def _compute_keys(pred_logits, pmt):
    return pl.pallas_call(
        _prob_body,
        grid=(B,),
        in_specs=[
            pl.BlockSpec((1, Q, TOK), lambda b: (b, 0, 0)),
            pl.BlockSpec((TOK, C), lambda b: (0, 0)),
        ],
        out_specs=pl.BlockSpec((1, Q, C), lambda b: (b, 0, 0)),
        out_shape=jax.ShapeDtypeStruct((B, Q, C), jnp.int32),
    )(pred_logits, pmt)


def _sc_scan_hist(hist_v, oth_v, run0):
    iota16 = lax.iota(jnp.int32, 16)

    def body(j, carry):
        run, found, bb, cgt = carry
        blk = NBV - 1 - j
        hv = hist_v[pl.ds(blk * 16, 16)] + oth_v[pl.ds(blk * 16, 16)]
        ssum = jnp.sum(hv)
        crossing = jnp.logical_and(found == 0, run + ssum >= K)

        def hit():
            rev = lax.rev(hv, (0,))
            incl = plsc.cumsum(rev)
            excl = incl - rev
            ge = (run + incl) >= K
            p = plsc.all_reduce_ffs(ge)
            p = jnp.max(p) if getattr(p, "ndim", 0) else p
            cg = run + jnp.sum(jnp.where(iota16 == p, excl, 0))
            return blk * 16 + 15 - p, cg

        bb2, cg2 = lax.cond(crossing, hit, lambda: (bb, cgt))
        found2 = jnp.where(crossing, 1, found)
        return (run + ssum, found2, bb2, cg2)

    _, _, bb, cgt = lax.fori_loop(0, NBV, body, (run0, 0, 0, 0))
    return bb, cgt


def _sc_zero(ref, n):
    z = jnp.zeros((16,), jnp.int32)

    def body(i, _):
        ref[pl.ds(i * 16, 16)] = z
        return 0

    lax.fori_loop(0, n // 16, body, 0)


@functools.cache
def _make_sc_select():
    mesh = plsc.VectorSubcoreMesh(core_axis_name="c", subcore_axis_name="s",
                                  num_cores=2, num_subcores=16)
    return functools.partial(
        pl.kernel,
        out_type=[
            jax.ShapeDtypeStruct((2 * B, CAND), jnp.int32),
            jax.ShapeDtypeStruct((2 * B, CAND), jnp.int32),
        ],
        mesh=mesh,
        compiler_params=pltpu.CompilerParams(needs_layout_passes=False),
        scratch_types=[
            pltpu.VMEM((HALF,), jnp.int32),
            pltpu.VMEM((NB,), jnp.int32),
            pltpu.VMEM((NB,), jnp.int32),
            pltpu.VMEM((CAND,), jnp.int32),
            pltpu.VMEM((CAND,), jnp.int32),
            pltpu.VMEM_SHARED((16, NB), jnp.int32),
        ],
    )(_sc_select_body)


def _sc_select_body(keys_hbm, ck_hbm, ci_hbm, keys_v, hist_v, oth_v, ck_v,
                    ci_v, sh_hist):
    c = lax.axis_index("c")
    s = lax.axis_index("s")
    r = c * 16 + s
    h = s % 2
    ones = jnp.ones((16,), jnp.int32)
    iota16 = lax.iota(jnp.int32, 16)

    pltpu.sync_copy(keys_hbm.at[r], keys_v)

    _sc_zero(hist_v, NB)

    def p1(i, _):
        v = keys_v[pl.ds(i * 16, 16)]
        plsc.addupdate_scatter(hist_v, [lax.shift_right_logical(v, 20)], ones)
        return 0

    lax.fori_loop(0, NV, p1, 0)

    pltpu.sync_copy(hist_v, sh_hist.at[s])
    plsc.subcore_barrier()
    pltpu.sync_copy(sh_hist.at[s ^ 1], oth_v)
    b0, cgt0 = _sc_scan_hist(hist_v, oth_v, 0)

    _sc_zero(hist_v, NB)

    def p2(i, _):
        v = keys_v[pl.ds(i * 16, 16)]
        m = lax.shift_right_logical(v, 20) == b0
        b2 = lax.shift_right_logical(v, 8) & 0xFFF
        plsc.addupdate_scatter(hist_v, [b2], ones, mask=m)
        return 0

    lax.fori_loop(0, NV, p2, 0)

    plsc.subcore_barrier()
    pltpu.sync_copy(hist_v, sh_hist.at[s])
    plsc.subcore_barrier()
    pltpu.sync_copy(sh_hist.at[s ^ 1], oth_v)
    b1, _ = _sc_scan_hist(hist_v, oth_v, cgt0)

    thresh = lax.shift_left(b0 * NB + b1, 8)

    _sc_zero(ck_v, CAND)
    _sc_zero(ci_v, CAND)
    idx_base = h * HALF

    def p3(i, cnt):
        v = keys_v[pl.ds(i * 16, 16)]
        m = v >= thresh
        mi = m.astype(jnp.int32)
        npos = jnp.sum(mi)

        @pl.when(npos > 0)
        def _():
            pos = cnt + plsc.cumsum(mi) - mi
            mm = jnp.logical_and(m, pos < CAND)
            plsc.store_scatter(ck_v, [pos], v, mask=mm)
            plsc.store_scatter(ci_v, [pos], idx_base + i * 16 + iota16,
                               mask=mm)

        return cnt + npos

    lax.fori_loop(0, NV, p3, 0)

    pltpu.sync_copy(ck_v, ck_hbm.at[r])
    pltpu.sync_copy(ci_v, ci_hbm.at[r])


def _order_body(ck_row_ref, ci_row_ref, ck_col_ref, ci_col_ref, boxes_ref,
                sf_ref, sc_ref, lb_ref, bx_ref):
    k_row = ck_row_ref[0]
    i_row = ci_row_ref[0]
    k_col = ck_col_ref[0]
    i_col = ci_col_ref[0]

    beats = jnp.logical_or(
        k_col > k_row,
        jnp.logical_and(k_col == k_row, i_col < i_row))
    rank = jnp.sum(beats.astype(jnp.int32), axis=0,
                   keepdims=True)

    riota = lax.broadcasted_iota(jnp.int32, (R, M), 0)
    oh = (riota == rank).astype(jnp.float32)

    prob_col = lax.bitcast_convert_type(k_col, jnp.float32)
    idxf_col = i_col.astype(jnp.float32)
    score_s = jnp.dot(oh, prob_col, precision=_HI,
                      preferred_element_type=jnp.float32)
    idxf_s = jnp.dot(oh, idxf_col, precision=_HI,
                     preferred_element_type=jnp.float32)
    idx_s = idxf_s.astype(jnp.int32)

    labels = idx_s % C
    qidx = idx_s // C

    qiota = lax.broadcasted_iota(jnp.int32, (R, Q), 1)
    oh2 = (qidx == qiota).astype(jnp.float32)
    bs = jnp.dot(oh2, boxes_ref[0], precision=_HI,
                 preferred_element_type=jnp.float32)

    cx, cy, w, h = bs[:, 0:1], bs[:, 1:2], bs[:, 2:3], bs[:, 3:4]
    xyxy = jnp.concatenate(
        [cx - 0.5 * w, cy - 0.5 * h, cx + 0.5 * w, cy + 0.5 * h], axis=1)
    xyxy = xyxy * sf_ref[0]

    sc_ref[0] = score_s[:K]
    lb_ref[0] = labels[:K]
    bx_ref[0] = xyxy[:K]


def _order(ck, ci, boxes, sf):
    return pl.pallas_call(
        _order_body,
        grid=(B,),
        in_specs=[
            pl.BlockSpec((1, 1, M), lambda b: (b, 0, 0)),
            pl.BlockSpec((1, 1, M), lambda b: (b, 0, 0)),
            pl.BlockSpec((1, M, 1), lambda b: (b, 0, 0)),
            pl.BlockSpec((1, M, 1), lambda b: (b, 0, 0)),
            pl.BlockSpec((1, Q, 4), lambda b: (b, 0, 0)),
            pl.BlockSpec((1, 1, 4), lambda b: (b, 0, 0)),
        ],
        out_specs=[
            pl.BlockSpec((1, K, 1), lambda b: (b, 0, 0)),
            pl.BlockSpec((1, K, 1), lambda b: (b, 0, 0)),
            pl.BlockSpec((1, K, 4), lambda b: (b, 0, 0)),
        ],
        out_shape=[
            jax.ShapeDtypeStruct((B, K, 1), jnp.float32),
            jax.ShapeDtypeStruct((B, K, 1), jnp.int32),
            jax.ShapeDtypeStruct((B, K, 4), jnp.float32),
        ],
    )(ck.reshape(B, 1, M), ci.reshape(B, 1, M), ck.reshape(B, M, 1),
      ci.reshape(B, M, 1), boxes, sf.reshape(B, 1, 4))


def kernel(pred_logits, pred_boxes, target_sizes, positive_map):
    keys = _compute_keys(pred_logits, positive_map.T)
    ck, ci = _make_sc_select()(keys.reshape(2 * B, HALF))
    ck = ck.reshape(B, M)
    ci = ci.reshape(B, M)
    ts = target_sizes.astype(jnp.float32)
    sf = jnp.stack([ts[:, 1], ts[:, 0], ts[:, 1], ts[:, 0]], axis=1)
    sc, lb, boxes = _order(ck, ci, pred_boxes, sf)
    return sc.reshape(B, K), lb.reshape(B, K), boxes


# speedup vs baseline: 3.9601x; 1.4108x over previous
import functools

import jax
import jax.numpy as jnp
from jax import lax
from jax.experimental import pallas as pl
from jax.experimental.pallas import tpu as pltpu
from jax.experimental.pallas import tpu_sc as plsc

B = 16
Q = 900
C = 80
TOK = 256
K = 300
HALF = Q * C // 2
NV = HALF // 16
NB = 4096
NBV = NB // 16
CAND = 384
M = 2 * CAND
R = 384

_HI = lax.Precision.HIGHEST


def _prob_body(logits_ref, pmt_ref, keys_ref):
    sig = jax.nn.sigmoid(logits_ref[0])
    prob = jnp.dot(sig, pmt_ref[...],
                   preferred_element_type=jnp.float32)
    keys_ref[0] = lax.bitcast_convert_type(prob, jnp.int32)


def _compute_keys(pred_logits, pmt):
    return pl.pallas_call(
        _prob_body,
        grid=(B,),
        in_specs=[
            pl.BlockSpec((1, Q, TOK), lambda b: (b, 0, 0)),
            pl.BlockSpec((TOK, C), lambda b: (0, 0)),
        ],
        out_specs=pl.BlockSpec((1, Q, C), lambda b: (b, 0, 0)),
        out_shape=jax.ShapeDtypeStruct((B, Q, C), jnp.int32),
    )(pred_logits, pmt)


def _sc_scan_hist(hist_v, oth_v, run0):
    iota16 = lax.iota(jnp.int32, 16)

    def body(j, carry):
        run, found, bb, cgt = carry
        blk = NBV - 1 - j
        hv = hist_v[pl.ds(blk * 16, 16)] + oth_v[pl.ds(blk * 16, 16)]
        ssum = jnp.sum(hv)
        crossing = jnp.logical_and(found == 0, run + ssum >= K)

        def hit():
            rev = lax.rev(hv, (0,))
            incl = plsc.cumsum(rev)
            excl = incl - rev
            ge = (run + incl) >= K
            p = plsc.all_reduce_ffs(ge)
            p = jnp.max(p) if getattr(p, "ndim", 0) else p
            cg = run + jnp.sum(jnp.where(iota16 == p, excl, 0))
            return blk * 16 + 15 - p, cg

        bb2, cg2 = lax.cond(crossing, hit, lambda: (bb, cgt))
        found2 = jnp.where(crossing, 1, found)
        return (run + ssum, found2, bb2, cg2)

    _, _, bb, cgt = lax.fori_loop(0, NBV, body, (run0, 0, 0, 0))
    return bb, cgt


def _sc_zero(ref, n):
    z = jnp.zeros((16,), jnp.int32)

    @plsc.parallel_loop(0, n // 16, 1, unroll=8)
    def _(i):
        ref[pl.ds(i * 16, 16)] = z


@functools.cache
def _make_sc_select():
    mesh = plsc.VectorSubcoreMesh(core_axis_name="c", subcore_axis_name="s",
                                  num_cores=2, num_subcores=16)
    return functools.partial(
        pl.kernel,
        out_type=[
            jax.ShapeDtypeStruct((2 * B, CAND), jnp.int32),
            jax.ShapeDtypeStruct((2 * B, CAND), jnp.int32),
        ],
        mesh=mesh,
        compiler_params=pltpu.CompilerParams(needs_layout_passes=False),
        scratch_types=[
            pltpu.VMEM((HALF,), jnp.int32),
            pltpu.VMEM((NB,), jnp.int32),
            pltpu.VMEM((NB,), jnp.int32),
            pltpu.VMEM((CAND,), jnp.int32),
            pltpu.VMEM((CAND,), jnp.int32),
            pltpu.VMEM_SHARED((16, NB), jnp.int32),
        ],
    )(_sc_select_body)


def _sc_select_body(keys_hbm, ck_hbm, ci_hbm, keys_v, hist_v, oth_v, ck_v,
                    ci_v, sh_hist):
    c = lax.axis_index("c")
    s = lax.axis_index("s")
    r = c * 16 + s
    h = s % 2
    ones = jnp.ones((16,), jnp.int32)
    iota16 = lax.iota(jnp.int32, 16)

    pltpu.sync_copy(keys_hbm.at[r], keys_v)

    _sc_zero(hist_v, NB)

    @plsc.parallel_loop(0, NV, 1, unroll=8)
    def _(i):
        v = keys_v[pl.ds(i * 16, 16)]
        plsc.addupdate_scatter(hist_v, [lax.shift_right_logical(v, 20)], ones)

    pltpu.sync_copy(hist_v, sh_hist.at[s])
    plsc.subcore_barrier()
    pltpu.sync_copy(sh_hist.at[s ^ 1], oth_v)
    b0, cgt0 = _sc_scan_hist(hist_v, oth_v, 0)

    _sc_zero(hist_v, NB)

    @plsc.parallel_loop(0, NV, 1, unroll=8)
    def _(i):
        v = keys_v[pl.ds(i * 16, 16)]
        m = lax.shift_right_logical(v, 20) == b0
        b2 = lax.shift_right_logical(v, 8) & 0xFFF
        plsc.addupdate_scatter(hist_v, [b2], ones, mask=m)

    plsc.subcore_barrier()
    pltpu.sync_copy(hist_v, sh_hist.at[s])
    plsc.subcore_barrier()
    pltpu.sync_copy(sh_hist.at[s ^ 1], oth_v)
    b1, _ = _sc_scan_hist(hist_v, oth_v, cgt0)

    thresh = lax.shift_left(b0 * NB + b1, 8)

    _sc_zero(ck_v, CAND)
    _sc_zero(ci_v, CAND)
    idx_base = h * HALF

    @plsc.parallel_loop(0, NV, 1, unroll=4, carry=jnp.int32(0))
    def _(i, cnt):
        v = keys_v[pl.ds(i * 16, 16)]
        m = v >= thresh
        mi = m.astype(jnp.int32)
        pos = cnt + plsc.cumsum(mi) - mi
        mm = jnp.logical_and(m, pos < CAND)
        plsc.store_scatter(ck_v, [pos], v, mask=mm)
        plsc.store_scatter(ci_v, [pos], idx_base + i * 16 + iota16, mask=mm)
        return cnt + jnp.sum(mi)

    pltpu.sync_copy(ck_v, ck_hbm.at[r])
    pltpu.sync_copy(ci_v, ci_hbm.at[r])


def _order_body(ck_row_ref, ci_row_ref, ck_col_ref, ci_col_ref, boxes_ref,
                sf_ref, sc_ref, lb_ref, bx_ref):
    k_row = ck_row_ref[0]
    i_row = ci_row_ref[0]
    k_col = ck_col_ref[0]
    i_col = ci_col_ref[0]

    beats = jnp.logical_or(
        k_col > k_row,
        jnp.logical_and(k_col == k_row, i_col < i_row))
    rank = jnp.sum(beats.astype(jnp.int32), axis=0,
                   keepdims=True)

    riota = lax.broadcasted_iota(jnp.int32, (R, M), 0)
    oh = (riota == rank).astype(jnp.float32)

    prob_col = lax.bitcast_convert_type(k_col, jnp.float32)
    idxf_col = i_col.astype(jnp.float32)
    score_s = jnp.dot(oh, prob_col, precision=_HI,
                      preferred_element_type=jnp.float32)
    idxf_s = jnp.dot(oh, idxf_col, precision=_HI,
                     preferred_element_type=jnp.float32)
    idx_s = idxf_s.astype(jnp.int32)

    labels = idx_s % C
    qidx = idx_s // C

    qiota = lax.broadcasted_iota(jnp.int32, (R, Q), 1)
    oh2 = (qidx == qiota).astype(jnp.float32)
    bs = jnp.dot(oh2, boxes_ref[0], precision=_HI,
                 preferred_element_type=jnp.float32)

    cx, cy, w, h = bs[:, 0:1], bs[:, 1:2], bs[:, 2:3], bs[:, 3:4]
    xyxy = jnp.concatenate(
        [cx - 0.5 * w, cy - 0.5 * h, cx + 0.5 * w, cy + 0.5 * h], axis=1)
    xyxy = xyxy * sf_ref[0]

    sc_ref[0] = score_s[:K]
    lb_ref[0] = labels[:K]
    bx_ref[0] = xyxy[:K]


def _order(ck, ci, boxes, sf):
    return pl.pallas_call(
        _order_body,
        grid=(B,),
        in_specs=[
            pl.BlockSpec((1, 1, M), lambda b: (b, 0, 0)),
            pl.BlockSpec((1, 1, M), lambda b: (b, 0, 0)),
            pl.BlockSpec((1, M, 1), lambda b: (b, 0, 0)),
            pl.BlockSpec((1, M, 1), lambda b: (b, 0, 0)),
            pl.BlockSpec((1, Q, 4), lambda b: (b, 0, 0)),
            pl.BlockSpec((1, 1, 4), lambda b: (b, 0, 0)),
        ],
        out_specs=[
            pl.BlockSpec((1, K, 1), lambda b: (b, 0, 0)),
            pl.BlockSpec((1, K, 1), lambda b: (b, 0, 0)),
            pl.BlockSpec((1, K, 4), lambda b: (b, 0, 0)),
        ],
        out_shape=[
            jax.ShapeDtypeStruct((B, K, 1), jnp.float32),
            jax.ShapeDtypeStruct((B, K, 1), jnp.int32),
            jax.ShapeDtypeStruct((B, K, 4), jnp.float32),
        ],
    )(ck.reshape(B, 1, M), ci.reshape(B, 1, M), ck.reshape(B, M, 1),
      ci.reshape(B, M, 1), boxes, sf.reshape(B, 1, 4))


def kernel(pred_logits, pred_boxes, target_sizes, positive_map):
    keys = _compute_keys(pred_logits, positive_map.T)
    ck, ci = _make_sc_select()(keys.reshape(2 * B, HALF))
    ck = ck.reshape(B, M)
    ci = ci.reshape(B, M)
    ts = target_sizes.astype(jnp.float32)
    sf = jnp.stack([ts[:, 1], ts[:, 0], ts[:, 1], ts[:, 0]], axis=1)
    sc, lb, boxes = _order(ck, ci, pred_boxes, sf)
    return sc.reshape(B, K), lb.reshape(B, K), boxes
